# baseline (speedup 1.0000x reference)
import functools

import jax
import jax.numpy as jnp
from jax import lax
from jax.experimental import pallas as pl
from jax.experimental.pallas import tpu as pltpu
from jax.experimental.pallas import tpu_sc as plsc

N = 10000
E = 320000
D = 128
DH = 64
NPAD = 10240
NB = 160
EPW = NB * 128
EPAD = 16 * EPW
BNEG = -1e9


def _head_body(x_ref, w_ref, asrc_ref, adst_ref, h_ref, as_ref, ad_ref):
    h = jnp.dot(x_ref[...], w_ref[...], preferred_element_type=jnp.float32)
    h_ref[0] = h[:, :DH]
    h_ref[1] = h[:, DH:]
    rid = pl.program_id(0) * 128 + lax.broadcasted_iota(jnp.int32, (128, 1), 0)
    valid = rid < N
    a_s = jnp.sum(h * asrc_ref[...], axis=1, keepdims=True)
    a_d = jnp.sum(h * adst_ref[...], axis=1, keepdims=True)
    as_ref[...] = jnp.where(valid, a_s, BNEG)
    ad_ref[...] = jnp.where(valid, a_d, BNEG)


def _head(x_pad, W, att_src, att_dst):
    return pl.pallas_call(
        _head_body,
        grid=(NPAD // 128,),
        in_specs=[
            pl.BlockSpec((128, D), lambda r: (r, 0)),
            pl.BlockSpec((D, D), lambda r: (0, 0)),
            pl.BlockSpec((1, D), lambda r: (0, 0)),
            pl.BlockSpec((1, D), lambda r: (0, 0)),
        ],
        out_specs=[
            pl.BlockSpec((2, 128, DH), lambda r: (0, r, 0)),
            pl.BlockSpec((128, 1), lambda r: (r, 0)),
            pl.BlockSpec((128, 1), lambda r: (r, 0)),
        ],
        out_shape=[
            jax.ShapeDtypeStruct((2, NPAD, DH), jnp.float32),
            jax.ShapeDtypeStruct((NPAD, 1), jnp.float32),
            jax.ShapeDtypeStruct((NPAD, 1), jnp.float32),
        ],
    )(x_pad, W, att_src, att_dst)


def _edge_body(se_hbm, as_hbm, ad_hbm, h_hbm, num_out, den_out,
               se_v, as_t, ad_t, si0, si1, si2, si3, di0, di1, di2, di3,
               sb0, sb1, rr0, rr1, rs0, rs1, acc_num, acc_den,
               sg0, sg1, sn0, sn1, sd0, sd1):
    c = lax.axis_index("c")
    s = lax.axis_index("s")
    zeros16 = jnp.zeros((16,), jnp.float32)
    rr = [rr0, rr1]
    rs = [rs0, rs1]
    sb = [sb0, sb1]
    si = [si0, si1, si2, si3]
    di = [di0, di1, di2, di3]
    sg = [sg0, sg1]
    sn = [sn0, sn1]
    sd = [sd0, sd1]

    def unpack(b, j):
        def upk(gg, _):
            sl = pl.ds(gg * 16, 16)
            p = se_v[b, sl]
            si[j][sl] = lax.bitwise_and(p, 0xFFFF)
            di[j][sl] = lax.shift_right_logical(p, 16)
            return 0
        lax.fori_loop(0, 8, upk, 0)

    def zrow(i, _):
        for k in range(DH // 16):
            rs0[i, pl.ds(k * 16, 16)] = zeros16
            rs1[i, pl.ds(k * 16, 16)] = zeros16
        return 0
    lax.fori_loop(0, 128, zrow, 0)
    for k in range(8):
        sb0[pl.ds(k * 16, 16)] = zeros16
        sb1[pl.ds(k * 16, 16)] = zeros16
    base = s * 640
    for i in range(5):
        pltpu.sync_copy(rs0, acc_num.at[pl.ds(base + i * 128, 128)])
        pltpu.sync_copy(sb0, acc_den.at[pl.ds(base + i * 128, 128)])

    pltpu.sync_copy(se_hbm.at[s], se_v)
    pltpu.sync_copy(as_hbm, as_t)
    pltpu.sync_copy(ad_hbm, ad_t)
    plsc.subcore_barrier()

    for k in range(4):
        unpack(k, k)
    for k in range(2):
        pltpu.async_copy(h_hbm.at[c].at[si[k]], rr[k], sg[k])
        pltpu.async_copy(rs[k], acc_num.at[di[0]], sn[k], add=True)
        pltpu.async_copy(sb[k], acc_den.at[di[0]], sd[k], add=True)

    def quad(g, _):
        for k in range(4):
            b = 4 * g + k
            s2 = k % 2
            j2 = (k + 2) % 4
            pltpu.make_async_copy(h_hbm.at[c].at[si[k]], rr[s2],
                                  sg[s2]).wait()
            pltpu.make_async_copy(rs[s2], acc_num.at[di[0]],
                                  sn[s2]).wait()
            pltpu.make_async_copy(sb[s2], acc_den.at[di[0]],
                                  sd[s2]).wait()
            bn = jnp.minimum(b + 2, NB - 1)
            unpack(bn, j2)
            def group(gg, _):
                sl = pl.ds(gg * 16, 16)
                p = se_v[b, sl]
                s_idx = lax.bitwise_and(p, 0xFFFF)
                d_idx = lax.shift_right_logical(p, 16)
                t = (plsc.load_gather(as_t, [s_idx])
                     + plsc.load_gather(ad_t, [d_idx]))
                t = jnp.where(t >= 0.0, t, 0.2 * t)
                sv_g = jnp.exp(t)
                sb[s2][sl] = sv_g
                j0 = gg * 16
                for l in range(16):
                    svl = lax.gather(
                        sv_g, jnp.full((16, 1), l, jnp.int32),
                        lax.GatherDimensionNumbers(
                            offset_dims=(), collapsed_slice_dims=(0,),
                            start_index_map=(0,)),
                        (1,),
                        mode=lax.GatherScatterMode.PROMISE_IN_BOUNDS)
                    for f in range(DH // 16):
                        rs[s2][j0 + l, pl.ds(f * 16, 16)] = (
                            rr[s2][j0 + l, pl.ds(f * 16, 16)] * svl)
                return 0
            lax.fori_loop(0, 8, group, 0)

            pltpu.async_copy(h_hbm.at[c].at[si[j2]], rr[s2], sg[s2])
            pltpu.async_copy(rs[s2], acc_num.at[di[k]], sn[s2], add=True)
            pltpu.async_copy(sb[s2], acc_den.at[di[k]], sd[s2], add=True)
        return 0
    lax.fori_loop(0, NB // 4, quad, 0)

    for k in range(2):
        pltpu.make_async_copy(h_hbm.at[c].at[si[k]], rr[k], sg[k]).wait()
        pltpu.make_async_copy(rs[k], acc_num.at[di[0]], sn[k]).wait()
        pltpu.make_async_copy(sb[k], acc_den.at[di[0]], sd[k]).wait()

    plsc.subcore_barrier()
    pltpu.sync_copy(acc_num.at[pl.ds(base, 640)],
                    num_out.at[c, pl.ds(base, 640)])
    pltpu.sync_copy(acc_den.at[pl.ds(base, 640)],
                    den_out.at[c, pl.ds(base, 640)])


_edge_call = functools.partial(
    pl.kernel,
    out_type=(jax.ShapeDtypeStruct((2, NPAD, DH), jnp.float32),
              jax.ShapeDtypeStruct((2, NPAD), jnp.float32)),
    mesh=plsc.VectorSubcoreMesh(core_axis_name="c", subcore_axis_name="s"),
    compiler_params=pltpu.CompilerParams(
        use_tc_tiling_on_sc=False, needs_layout_passes=False),
    scratch_types=[
        pltpu.VMEM((NB, 128), jnp.int32),
        pltpu.VMEM((NPAD,), jnp.float32),
        pltpu.VMEM((NPAD,), jnp.float32),
        pltpu.VMEM((128,), jnp.int32),
        pltpu.VMEM((128,), jnp.int32),
        pltpu.VMEM((128,), jnp.int32),
        pltpu.VMEM((128,), jnp.int32),
        pltpu.VMEM((128,), jnp.int32),
        pltpu.VMEM((128,), jnp.int32),
        pltpu.VMEM((128,), jnp.int32),
        pltpu.VMEM((128,), jnp.int32),
        pltpu.VMEM((128,), jnp.float32),
        pltpu.VMEM((128,), jnp.float32),
        pltpu.VMEM((128, DH), jnp.float32),
        pltpu.VMEM((128, DH), jnp.float32),
        pltpu.VMEM((128, DH), jnp.float32),
        pltpu.VMEM((128, DH), jnp.float32),
        pltpu.VMEM_SHARED((NPAD, DH), jnp.float32),
        pltpu.VMEM_SHARED((NPAD,), jnp.float32),
    ] + [pltpu.SemaphoreType.DMA] * 6,
)(_edge_body)


def _tail_body(num_ref, den_ref, h_ref, asrc_ref, adst_ref, b_ref, a_ref,
               o_ref):
    h = jnp.concatenate([h_ref[0], h_ref[1]], axis=1)
    t = (jnp.sum(h * asrc_ref[...], axis=1, keepdims=True)
         + jnp.sum(h * adst_ref[...], axis=1, keepdims=True))
    w_self = jnp.exp(jnp.where(t >= 0.0, t, 0.2 * t))
    num = jnp.concatenate([num_ref[0], num_ref[1]], axis=1) + w_self * h
    den = den_ref[0] + w_self + 1e-16
    out = num / den + b_ref[...]
    a = a_ref[0, 0]
    o_ref[...] = jnp.where(out >= 0.0, out, a * out)


def _tail(num, den, h3, att_src, att_dst, bias, a):
    return pl.pallas_call(
        _tail_body,
        grid=(NPAD // 128,),
        in_specs=[
            pl.BlockSpec((2, 128, DH), lambda r: (0, r, 0)),
            pl.BlockSpec((2, 128, 1), lambda r: (0, r, 0)),
            pl.BlockSpec((2, 128, DH), lambda r: (0, r, 0)),
            pl.BlockSpec((1, D), lambda r: (0, 0)),
            pl.BlockSpec((1, D), lambda r: (0, 0)),
            pl.BlockSpec((1, D), lambda r: (0, 0)),
            pl.BlockSpec((1, 1), lambda r: (0, 0)),
        ],
        out_specs=pl.BlockSpec((128, D), lambda r: (r, 0)),
        out_shape=jax.ShapeDtypeStruct((NPAD, D), jnp.float32),
    )(num, den, h3, att_src, att_dst, bias, a)


def _layer(x_pad, se_g, W, att_src, att_dst, bias, a):
    asr = att_src.reshape(1, D)
    adr = att_dst.reshape(1, D)
    h3, a_s, a_d = _head(x_pad, W, asr, adr)
    num, den = _edge_call(se_g, a_s.reshape(NPAD), a_d.reshape(NPAD), h3)
    return _tail(num, den.reshape(2, NPAD, 1), h3, asr, adr,
                 bias.reshape(1, D), a.reshape(1, 1))


def kernel(x, edge_index, W1, att_src1, att_dst1, bias1, a1,
           W2, att_src2, att_dst2, bias2, a2):
    pad_idx = N + (jnp.arange(EPAD - E, dtype=jnp.int32) % 16)
    src_g = jnp.concatenate([edge_index[0], pad_idx])
    dst_g = jnp.concatenate([edge_index[1], pad_idx])
    se_g = (src_g | (dst_g << 16)).reshape(16, NB, 128)
    x_pad = jnp.pad(x, ((0, NPAD - N), (0, 0)))
    x2 = _layer(x_pad, se_g, W1, att_src1, att_dst1, bias1, a1)
    out = _layer(x2, se_g, W2, att_src2, att_dst2, bias2, a2)
    return out[:N]


# speedup vs baseline: 1.0641x; 1.0641x over previous
import functools

import jax
import jax.numpy as jnp
from jax import lax
from jax.experimental import pallas as pl
from jax.experimental.pallas import tpu as pltpu
from jax.experimental.pallas import tpu_sc as plsc

N = 10000
E = 320000
D = 128
DH = 64
NPAD = 10240
NB = 158
EPW = NB * 128
EPAD = 16 * EPW
BNEG = -1e9


def _head_body(x_ref, w_ref, asrc_ref, adst_ref, h_ref, as_ref, ad_ref):
    h = jnp.dot(x_ref[...], w_ref[...], preferred_element_type=jnp.float32)
    h_ref[0] = h[:, :DH]
    h_ref[1] = h[:, DH:]
    rid = pl.program_id(0) * 128 + lax.broadcasted_iota(jnp.int32, (128, 1), 0)
    valid = rid < N
    a_s = jnp.sum(h * asrc_ref[...], axis=1, keepdims=True)
    a_d = jnp.sum(h * adst_ref[...], axis=1, keepdims=True)
    as_ref[...] = jnp.where(valid, a_s, BNEG)
    ad_ref[...] = jnp.where(valid, a_d, BNEG)


def _head(x_pad, W, att_src, att_dst):
    return pl.pallas_call(
        _head_body,
        grid=(NPAD // 128,),
        in_specs=[
            pl.BlockSpec((128, D), lambda r: (r, 0)),
            pl.BlockSpec((D, D), lambda r: (0, 0)),
            pl.BlockSpec((1, D), lambda r: (0, 0)),
            pl.BlockSpec((1, D), lambda r: (0, 0)),
        ],
        out_specs=[
            pl.BlockSpec((2, 128, DH), lambda r: (0, r, 0)),
            pl.BlockSpec((128, 1), lambda r: (r, 0)),
            pl.BlockSpec((128, 1), lambda r: (r, 0)),
        ],
        out_shape=[
            jax.ShapeDtypeStruct((2, NPAD, DH), jnp.float32),
            jax.ShapeDtypeStruct((NPAD, 1), jnp.float32),
            jax.ShapeDtypeStruct((NPAD, 1), jnp.float32),
        ],
    )(x_pad, W, att_src, att_dst)


def _edge_body(src_hbm, dst_hbm, as_hbm, ad_hbm, h_hbm, num_out, den_out,
               src_v, dst_v, as0, as1, ad0, ad1, sb0, sb1, rr0, rr1, rs0, rs1,
               acc_num, acc_den,
               sg0, sg1, sa0, sa1, sc0, sc1, sn0, sn1, sd0, sd1):
    c = lax.axis_index("c")
    s = lax.axis_index("s")
    zeros16 = jnp.zeros((16,), jnp.float32)
    asg = [as0, as1]
    adg = [ad0, ad1]
    sb = [sb0, sb1]
    rr = [rr0, rr1]
    rs = [rs0, rs1]
    sg = [sg0, sg1]
    sa = [sa0, sa1]
    sc = [sc0, sc1]
    sn = [sn0, sn1]
    sd = [sd0, sd1]

    def zrow(i, _):
        for k in range(DH // 16):
            rs0[i, pl.ds(k * 16, 16)] = zeros16
            rs1[i, pl.ds(k * 16, 16)] = zeros16
        return 0
    lax.fori_loop(0, 128, zrow, 0)
    for k in range(8):
        sb0[pl.ds(k * 16, 16)] = zeros16
        sb1[pl.ds(k * 16, 16)] = zeros16
    base = s * 640
    for i in range(5):
        pltpu.sync_copy(rs0, acc_num.at[pl.ds(base + i * 128, 128)])
        pltpu.sync_copy(sb0, acc_den.at[pl.ds(base + i * 128, 128)])

    pltpu.sync_copy(src_hbm.at[s], src_v)
    pltpu.sync_copy(dst_hbm.at[s], dst_v)
    plsc.subcore_barrier()

    for k in range(2):
        pltpu.async_copy(h_hbm.at[c].at[src_v.at[k]], rr[k], sg[k])
        pltpu.async_copy(as_hbm.at[src_v.at[k]], asg[k], sa[k])
        pltpu.async_copy(ad_hbm.at[dst_v.at[k]], adg[k], sc[k])
        pltpu.async_copy(rs[k], acc_num.at[dst_v.at[0]], sn[k], add=True)
        pltpu.async_copy(sb[k], acc_den.at[dst_v.at[0]], sd[k], add=True)

    def pair(g, _):
        for k in range(2):
            b = 2 * g + k
            pltpu.make_async_copy(h_hbm.at[c].at[src_v.at[b]], rr[k],
                                  sg[k]).wait()
            pltpu.make_async_copy(as_hbm.at[src_v.at[b]], asg[k],
                                  sa[k]).wait()
            pltpu.make_async_copy(ad_hbm.at[dst_v.at[b]], adg[k],
                                  sc[k]).wait()
            pltpu.make_async_copy(rs[k], acc_num.at[dst_v.at[0]],
                                  sn[k]).wait()
            pltpu.make_async_copy(sb[k], acc_den.at[dst_v.at[0]],
                                  sd[k]).wait()
            def group(gg, _):
                sl = pl.ds(gg * 16, 16)
                t = asg[k][sl] + adg[k][sl]
                t = jnp.where(t >= 0.0, t, 0.2 * t)
                sv_g = jnp.exp(t)
                sb[k][sl] = sv_g
                j0 = gg * 16
                for l in range(16):
                    svl = lax.gather(
                        sv_g, jnp.full((16, 1), l, jnp.int32),
                        lax.GatherDimensionNumbers(
                            offset_dims=(), collapsed_slice_dims=(0,),
                            start_index_map=(0,)),
                        (1,),
                        mode=lax.GatherScatterMode.PROMISE_IN_BOUNDS)
                    for f in range(DH // 16):
                        rs[k][j0 + l, pl.ds(f * 16, 16)] = (
                            rr[k][j0 + l, pl.ds(f * 16, 16)] * svl)
                return 0
            lax.fori_loop(0, 8, group, 0)

            bn = jnp.minimum(b + 2, NB - 1)
            pltpu.async_copy(h_hbm.at[c].at[src_v.at[bn]], rr[k], sg[k])
            pltpu.async_copy(as_hbm.at[src_v.at[bn]], asg[k], sa[k])
            pltpu.async_copy(ad_hbm.at[dst_v.at[bn]], adg[k], sc[k])
            pltpu.async_copy(rs[k], acc_num.at[dst_v.at[b]], sn[k], add=True)
            pltpu.async_copy(sb[k], acc_den.at[dst_v.at[b]], sd[k], add=True)
        return 0
    lax.fori_loop(0, NB // 2, pair, 0)

    for k in range(2):
        pltpu.make_async_copy(h_hbm.at[c].at[src_v.at[0]], rr[k],
                              sg[k]).wait()
        pltpu.make_async_copy(as_hbm.at[src_v.at[0]], asg[k], sa[k]).wait()
        pltpu.make_async_copy(ad_hbm.at[dst_v.at[0]], adg[k], sc[k]).wait()
        pltpu.make_async_copy(rs[k], acc_num.at[dst_v.at[0]], sn[k]).wait()
        pltpu.make_async_copy(sb[k], acc_den.at[dst_v.at[0]], sd[k]).wait()

    plsc.subcore_barrier()
    pltpu.sync_copy(acc_num.at[pl.ds(base, 640)],
                    num_out.at[c, pl.ds(base, 640)])
    pltpu.sync_copy(acc_den.at[pl.ds(base, 640)],
                    den_out.at[c, pl.ds(base, 640)])


_edge_call = functools.partial(
    pl.kernel,
    out_type=(jax.ShapeDtypeStruct((2, NPAD, DH), jnp.float32),
              jax.ShapeDtypeStruct((2, NPAD), jnp.float32)),
    mesh=plsc.VectorSubcoreMesh(core_axis_name="c", subcore_axis_name="s"),
    compiler_params=pltpu.CompilerParams(
        use_tc_tiling_on_sc=False, needs_layout_passes=False),
    scratch_types=[
        pltpu.VMEM((NB, 128), jnp.int32),
        pltpu.VMEM((NB, 128), jnp.int32),
        pltpu.VMEM((128,), jnp.float32),
        pltpu.VMEM((128,), jnp.float32),
        pltpu.VMEM((128,), jnp.float32),
        pltpu.VMEM((128,), jnp.float32),
        pltpu.VMEM((128,), jnp.float32),
        pltpu.VMEM((128,), jnp.float32),
        pltpu.VMEM((128, DH), jnp.float32),
        pltpu.VMEM((128, DH), jnp.float32),
        pltpu.VMEM((128, DH), jnp.float32),
        pltpu.VMEM((128, DH), jnp.float32),
        pltpu.VMEM_SHARED((NPAD, DH), jnp.float32),
        pltpu.VMEM_SHARED((NPAD,), jnp.float32),
    ] + [pltpu.SemaphoreType.DMA] * 10,
)(_edge_body)


def _tail_body(num_ref, den_ref, h_ref, asrc_ref, adst_ref, b_ref, a_ref,
               o_ref):
    h = jnp.concatenate([h_ref[0], h_ref[1]], axis=1)
    t = (jnp.sum(h * asrc_ref[...], axis=1, keepdims=True)
         + jnp.sum(h * adst_ref[...], axis=1, keepdims=True))
    w_self = jnp.exp(jnp.where(t >= 0.0, t, 0.2 * t))
    num = jnp.concatenate([num_ref[0], num_ref[1]], axis=1) + w_self * h
    den = den_ref[0] + w_self + 1e-16
    out = num / den + b_ref[...]
    a = a_ref[0, 0]
    o_ref[...] = jnp.where(out >= 0.0, out, a * out)


def _tail(num, den, h3, att_src, att_dst, bias, a):
    return pl.pallas_call(
        _tail_body,
        grid=(NPAD // 128,),
        in_specs=[
            pl.BlockSpec((2, 128, DH), lambda r: (0, r, 0)),
            pl.BlockSpec((2, 128, 1), lambda r: (0, r, 0)),
            pl.BlockSpec((2, 128, DH), lambda r: (0, r, 0)),
            pl.BlockSpec((1, D), lambda r: (0, 0)),
            pl.BlockSpec((1, D), lambda r: (0, 0)),
            pl.BlockSpec((1, D), lambda r: (0, 0)),
            pl.BlockSpec((1, 1), lambda r: (0, 0)),
        ],
        out_specs=pl.BlockSpec((128, D), lambda r: (r, 0)),
        out_shape=jax.ShapeDtypeStruct((NPAD, D), jnp.float32),
    )(num, den, h3, att_src, att_dst, bias, a)


def _layer(x_pad, src_g, dst_g, W, att_src, att_dst, bias, a):
    asr = att_src.reshape(1, D)
    adr = att_dst.reshape(1, D)
    h3, a_s, a_d = _head(x_pad, W, asr, adr)
    num, den = _edge_call(src_g, dst_g, a_s.reshape(NPAD), a_d.reshape(NPAD),
                          h3)
    return _tail(num, den.reshape(2, NPAD, 1), h3, asr, adr,
                 bias.reshape(1, D), a.reshape(1, 1))


def kernel(x, edge_index, W1, att_src1, att_dst1, bias1, a1,
           W2, att_src2, att_dst2, bias2, a2):
    pad_idx = N + (jnp.arange(EPAD - E, dtype=jnp.int32) % 16)
    src_g = jnp.concatenate([edge_index[0], pad_idx]).reshape(16, NB, 128)
    dst_g = jnp.concatenate([edge_index[1], pad_idx]).reshape(16, NB, 128)
    x_pad = jnp.pad(x, ((0, NPAD - N), (0, 0)))
    x2 = _layer(x_pad, src_g, dst_g, W1, att_src1, att_dst1, bias1, a1)
    out = _layer(x2, src_g, dst_g, W2, att_src2, att_dst2, bias2, a2)
    return out[:N]


# speedup vs baseline: 1.1025x; 1.0361x over previous
import functools

import jax
import jax.numpy as jnp
from jax import lax
from jax.experimental import pallas as pl
from jax.experimental.pallas import tpu as pltpu
from jax.experimental.pallas import tpu_sc as plsc

N = 10000
E = 320000
D = 128
DH = 64
NPAD = 10240
NB = 158
EPW = NB * 128
EPAD = 16 * EPW
BNEG = -1e9


def _head_body(x_ref, w_ref, asrc_ref, adst_ref, h_ref, as_ref, ad_ref):
    h = jnp.dot(x_ref[...], w_ref[...], preferred_element_type=jnp.float32)
    h_ref[0] = h[:, :DH]
    h_ref[1] = h[:, DH:]
    rid = pl.program_id(0) * 128 + lax.broadcasted_iota(jnp.int32, (128, 1), 0)
    valid = rid < N
    a_s = jnp.sum(h * asrc_ref[...], axis=1, keepdims=True)
    a_d = jnp.sum(h * adst_ref[...], axis=1, keepdims=True)
    as_ref[...] = jnp.where(valid, a_s, BNEG)
    ad_ref[...] = jnp.where(valid, a_d, BNEG)


def _head(x_pad, W, att_src, att_dst):
    return pl.pallas_call(
        _head_body,
        grid=(NPAD // 128,),
        in_specs=[
            pl.BlockSpec((128, D), lambda r: (r, 0)),
            pl.BlockSpec((D, D), lambda r: (0, 0)),
            pl.BlockSpec((1, D), lambda r: (0, 0)),
            pl.BlockSpec((1, D), lambda r: (0, 0)),
        ],
        out_specs=[
            pl.BlockSpec((2, 128, DH), lambda r: (0, r, 0)),
            pl.BlockSpec((128, 1), lambda r: (r, 0)),
            pl.BlockSpec((128, 1), lambda r: (r, 0)),
        ],
        out_shape=[
            jax.ShapeDtypeStruct((2, NPAD, DH), jnp.float32),
            jax.ShapeDtypeStruct((NPAD, 1), jnp.float32),
            jax.ShapeDtypeStruct((NPAD, 1), jnp.float32),
        ],
    )(x_pad, W, att_src, att_dst)


def _edge_body(src_hbm, dst_hbm, as_hbm, ad_hbm, h_hbm, num_out, den_out,
               src_v, dst_v, as0, as1, ad0, ad1, sb0, sb1, rr0, rr1, rs0, rs1,
               acc_num, acc_den,
               sg0, sg1, sa0, sa1, sc0, sc1, sn0, sn1, sd0, sd1):
    c = lax.axis_index("c")
    s = lax.axis_index("s")
    zeros16 = jnp.zeros((16,), jnp.float32)
    asg = [as0, as1]
    adg = [ad0, ad1]
    sb = [sb0, sb1]
    rr = [rr0, rr1]
    rs = [rs0, rs1]
    sg = [sg0, sg1]
    sa = [sa0, sa1]
    sc = [sc0, sc1]
    sn = [sn0, sn1]
    sd = [sd0, sd1]

    def zrow(i, _):
        for k in range(DH // 16):
            rs0[i, pl.ds(k * 16, 16)] = zeros16
            rs1[i, pl.ds(k * 16, 16)] = zeros16
        return 0
    lax.fori_loop(0, 128, zrow, 0)
    for k in range(8):
        sb0[pl.ds(k * 16, 16)] = zeros16
        sb1[pl.ds(k * 16, 16)] = zeros16
    base = s * 640
    for i in range(5):
        pltpu.sync_copy(rs0, acc_num.at[pl.ds(base + i * 128, 128)])
        pltpu.sync_copy(sb0, acc_den.at[pl.ds(base + i * 128, 128)])

    pltpu.sync_copy(src_hbm.at[s], src_v)
    pltpu.sync_copy(dst_hbm.at[s], dst_v)
    plsc.subcore_barrier()

    for k in range(2):
        pltpu.async_copy(h_hbm.at[c].at[src_v.at[k]], rr[k], sg[k])
        pltpu.async_copy(as_hbm.at[src_v.at[k]], asg[k], sa[k])
        pltpu.async_copy(ad_hbm.at[dst_v.at[k]], adg[k], sc[k])
        pltpu.async_copy(rs[k], acc_num.at[dst_v.at[0]], sn[k], add=True)
        pltpu.async_copy(sb[k], acc_den.at[dst_v.at[0]], sd[k], add=True)

    def pair(g, _):
        for k in range(2):
            b = 2 * g + k
            pltpu.make_async_copy(h_hbm.at[c].at[src_v.at[b]], rr[k],
                                  sg[k]).wait()
            pltpu.make_async_copy(as_hbm.at[src_v.at[b]], asg[k],
                                  sa[k]).wait()
            pltpu.make_async_copy(ad_hbm.at[dst_v.at[b]], adg[k],
                                  sc[k]).wait()
            pltpu.make_async_copy(rs[k], acc_num.at[dst_v.at[0]],
                                  sn[k]).wait()
            pltpu.make_async_copy(sb[k], acc_den.at[dst_v.at[0]],
                                  sd[k]).wait()
            def group(gg, _):
                sl = pl.ds(gg * 16, 16)
                t = asg[k][sl] + adg[k][sl]
                t = jnp.where(t >= 0.0, t, 0.2 * t)
                sv_g = jnp.exp(t)
                sb[k][sl] = sv_g
                j0 = gg * 16
                for l in range(16):
                    svl = lax.gather(
                        sv_g, jnp.full((16, 1), l, jnp.int32),
                        lax.GatherDimensionNumbers(
                            offset_dims=(), collapsed_slice_dims=(0,),
                            start_index_map=(0,)),
                        (1,),
                        mode=lax.GatherScatterMode.PROMISE_IN_BOUNDS)
                    for f in range(DH // 16):
                        rs[k][j0 + l, pl.ds(f * 16, 16)] = (
                            rr[k][j0 + l, pl.ds(f * 16, 16)] * svl)
                return 0
            lax.fori_loop(0, 8, group, 0)

            bn = jnp.minimum(b + 2, NB - 1)
            pltpu.async_copy(h_hbm.at[c].at[src_v.at[bn]], rr[k], sg[k])
            pltpu.async_copy(as_hbm.at[src_v.at[bn]], asg[k], sa[k])
            pltpu.async_copy(ad_hbm.at[dst_v.at[bn]], adg[k], sc[k])
            pltpu.async_copy(rs[k], acc_num.at[dst_v.at[b]], sn[k], add=True)
            pltpu.async_copy(sb[k], acc_den.at[dst_v.at[b]], sd[k], add=True)
        return 0
    lax.fori_loop(0, NB // 2, pair, 0)

    for k in range(2):
        pltpu.make_async_copy(h_hbm.at[c].at[src_v.at[0]], rr[k],
                              sg[k]).wait()
        pltpu.make_async_copy(as_hbm.at[src_v.at[0]], asg[k], sa[k]).wait()
        pltpu.make_async_copy(ad_hbm.at[dst_v.at[0]], adg[k], sc[k]).wait()
        pltpu.make_async_copy(rs[k], acc_num.at[dst_v.at[0]], sn[k]).wait()
        pltpu.make_async_copy(sb[k], acc_den.at[dst_v.at[0]], sd[k]).wait()

    plsc.subcore_barrier()
    pltpu.sync_copy(acc_num.at[pl.ds(base, 640)],
                    num_out.at[c, pl.ds(base, 640)])
    pltpu.sync_copy(acc_den.at[pl.ds(base, 640)],
                    den_out.at[c, pl.ds(base, 640)])


_edge_call = functools.partial(
    pl.kernel,
    out_type=(jax.ShapeDtypeStruct((2, NPAD, DH), jnp.float32),
              jax.ShapeDtypeStruct((2, NPAD), jnp.float32)),
    mesh=plsc.VectorSubcoreMesh(core_axis_name="c", subcore_axis_name="s"),
    compiler_params=pltpu.CompilerParams(
        use_tc_tiling_on_sc=False, needs_layout_passes=False),
    scratch_types=[
        pltpu.VMEM((NB, 128), jnp.int32),
        pltpu.VMEM((NB, 128), jnp.int32),
        pltpu.VMEM((128,), jnp.float32),
        pltpu.VMEM((128,), jnp.float32),
        pltpu.VMEM((128,), jnp.float32),
        pltpu.VMEM((128,), jnp.float32),
        pltpu.VMEM((128,), jnp.float32),
        pltpu.VMEM((128,), jnp.float32),
        pltpu.VMEM((128, DH), jnp.float32),
        pltpu.VMEM((128, DH), jnp.float32),
        pltpu.VMEM((128, DH), jnp.float32),
        pltpu.VMEM((128, DH), jnp.float32),
        pltpu.VMEM_SHARED((NPAD, DH), jnp.float32),
        pltpu.VMEM_SHARED((NPAD,), jnp.float32),
    ] + [pltpu.SemaphoreType.DMA] * 10,
)(_edge_body)


def _mid_body(num_ref, den_ref, h_ref, as1_ref, ad1_ref, b1_ref, a1_ref,
              w2_ref, as2_ref, ad2_ref, h2_ref, aso_ref, ado_ref):
    h = jnp.concatenate([h_ref[0], h_ref[1]], axis=1)
    t = (jnp.sum(h * as1_ref[...], axis=1, keepdims=True)
         + jnp.sum(h * ad1_ref[...], axis=1, keepdims=True))
    w_self = jnp.exp(jnp.where(t >= 0.0, t, 0.2 * t))
    num = jnp.concatenate([num_ref[0], num_ref[1]], axis=1) + w_self * h
    den = den_ref[0] + w_self + 1e-16
    x2 = num / den + b1_ref[...]
    a = a1_ref[0, 0]
    x2 = jnp.where(x2 >= 0.0, x2, a * x2)
    h2 = jnp.dot(x2, w2_ref[...], preferred_element_type=jnp.float32)
    h2_ref[0] = h2[:, :DH]
    h2_ref[1] = h2[:, DH:]
    rid = pl.program_id(0) * 128 + lax.broadcasted_iota(jnp.int32, (128, 1), 0)
    valid = rid < N
    a_s = jnp.sum(h2 * as2_ref[...], axis=1, keepdims=True)
    a_d = jnp.sum(h2 * ad2_ref[...], axis=1, keepdims=True)
    aso_ref[...] = jnp.where(valid, a_s, BNEG)
    ado_ref[...] = jnp.where(valid, a_d, BNEG)


def _mid(num, den, h3, as1, ad1, bias1, a1, W2, as2, ad2):
    return pl.pallas_call(
        _mid_body,
        grid=(NPAD // 128,),
        in_specs=[
            pl.BlockSpec((2, 128, DH), lambda r: (0, r, 0)),
            pl.BlockSpec((2, 128, 1), lambda r: (0, r, 0)),
            pl.BlockSpec((2, 128, DH), lambda r: (0, r, 0)),
            pl.BlockSpec((1, D), lambda r: (0, 0)),
            pl.BlockSpec((1, D), lambda r: (0, 0)),
            pl.BlockSpec((1, D), lambda r: (0, 0)),
            pl.BlockSpec((1, 1), lambda r: (0, 0)),
            pl.BlockSpec((D, D), lambda r: (0, 0)),
            pl.BlockSpec((1, D), lambda r: (0, 0)),
            pl.BlockSpec((1, D), lambda r: (0, 0)),
        ],
        out_specs=[
            pl.BlockSpec((2, 128, DH), lambda r: (0, r, 0)),
            pl.BlockSpec((128, 1), lambda r: (r, 0)),
            pl.BlockSpec((128, 1), lambda r: (r, 0)),
        ],
        out_shape=[
            jax.ShapeDtypeStruct((2, NPAD, DH), jnp.float32),
            jax.ShapeDtypeStruct((NPAD, 1), jnp.float32),
            jax.ShapeDtypeStruct((NPAD, 1), jnp.float32),
        ],
    )(num, den, h3, as1, ad1, bias1, a1, W2, as2, ad2)


def _tail_body(num_ref, den_ref, h_ref, asrc_ref, adst_ref, b_ref, a_ref,
               o_ref):
    h = jnp.concatenate([h_ref[0], h_ref[1]], axis=1)
    t = (jnp.sum(h * asrc_ref[...], axis=1, keepdims=True)
         + jnp.sum(h * adst_ref[...], axis=1, keepdims=True))
    w_self = jnp.exp(jnp.where(t >= 0.0, t, 0.2 * t))
    num = jnp.concatenate([num_ref[0], num_ref[1]], axis=1) + w_self * h
    den = den_ref[0] + w_self + 1e-16
    out = num / den + b_ref[...]
    a = a_ref[0, 0]
    o_ref[...] = jnp.where(out >= 0.0, out, a * out)


def _tail(num, den, h3, att_src, att_dst, bias, a):
    return pl.pallas_call(
        _tail_body,
        grid=(NPAD // 128,),
        in_specs=[
            pl.BlockSpec((2, 128, DH), lambda r: (0, r, 0)),
            pl.BlockSpec((2, 128, 1), lambda r: (0, r, 0)),
            pl.BlockSpec((2, 128, DH), lambda r: (0, r, 0)),
            pl.BlockSpec((1, D), lambda r: (0, 0)),
            pl.BlockSpec((1, D), lambda r: (0, 0)),
            pl.BlockSpec((1, D), lambda r: (0, 0)),
            pl.BlockSpec((1, 1), lambda r: (0, 0)),
        ],
        out_specs=pl.BlockSpec((128, D), lambda r: (r, 0)),
        out_shape=jax.ShapeDtypeStruct((NPAD, D), jnp.float32),
    )(num, den, h3, att_src, att_dst, bias, a)


def kernel(x, edge_index, W1, att_src1, att_dst1, bias1, a1,
           W2, att_src2, att_dst2, bias2, a2):
    pad_idx = N + (jnp.arange(EPAD - E, dtype=jnp.int32) % 16)
    src_g = jnp.concatenate([edge_index[0], pad_idx]).reshape(16, NB, 128)
    dst_g = jnp.concatenate([edge_index[1], pad_idx]).reshape(16, NB, 128)
    x_pad = jnp.pad(x, ((0, NPAD - N), (0, 0)))
    as1 = att_src1.reshape(1, D)
    ad1 = att_dst1.reshape(1, D)
    as2 = att_src2.reshape(1, D)
    ad2 = att_dst2.reshape(1, D)
    h1, a_s1, a_d1 = _head(x_pad, W1, as1, ad1)
    num1, den1 = _edge_call(src_g, dst_g, a_s1.reshape(NPAD),
                            a_d1.reshape(NPAD), h1)
    h2, a_s2, a_d2 = _mid(num1, den1.reshape(2, NPAD, 1), h1, as1, ad1,
                          bias1.reshape(1, D), a1.reshape(1, 1), W2, as2, ad2)
    num2, den2 = _edge_call(src_g, dst_g, a_s2.reshape(NPAD),
                            a_d2.reshape(NPAD), h2)
    out = _tail(num2, den2.reshape(2, NPAD, 1), h2, as2, ad2,
                bias2.reshape(1, D), a2.reshape(1, 1))
    return out[:N]
